# baseline (speedup 1.0000x reference)
import functools

import jax
import jax.numpy as jnp
from jax import lax
from jax.experimental import pallas as pl
from jax.experimental.pallas import tpu as pltpu
from jax.experimental.pallas import tpu_sc as plsc

N_TOKEN = 1000000
D = 64
SCALE = 8.0
LANES = 16

B_TOTAL = 4096 * 200
NW = 32
PER_W = B_TOTAL // NW
CH = 512
GATHER_W = 128
K = CH // GATHER_W
N_CHUNKS = PER_W // CH


def _emb_kernel(tok_hbm, table_hbm, out_hbm, idx_v, rows_v, scale_v, sem):
    nc = 2
    wid = lax.axis_index("s") * nc + lax.axis_index("c")
    base = wid * PER_W

    def chunk_body(g, carry):
        row0 = base + g * CH
        pltpu.sync_copy(tok_hbm.at[pl.ds(row0, CH)], idx_v)
        cps = []
        for j in range(K):
            cps.append(pltpu.async_copy(
                table_hbm.at[idx_v.at[pl.ds(j * GATHER_W, GATHER_W)]],
                rows_v.at[pl.ds(j * GATHER_W, GATHER_W)],
                sem))
        for cp in cps:
            cp.wait()
        for q in range(CH // LANES):
            t = idx_v[pl.ds(q * LANES, LANES)]
            scale_v[pl.ds(q * LANES, LANES)] = jnp.where(
                t == 0, jnp.float32(0.0), jnp.float32(SCALE))

        def mul_body(r, c2):
            sv = scale_v[pl.ds(r, LANES)]
            sp = jnp.full((LANES,), sv[0], jnp.float32)
            for c in range(D // LANES):
                x = rows_v[r, pl.ds(c * LANES, LANES)]
                rows_v[r, pl.ds(c * LANES, LANES)] = x * sp
            return c2

        lax.fori_loop(0, CH, mul_body, 0)
        pltpu.sync_copy(rows_v, out_hbm.at[pl.ds(row0, CH)])
        return carry

    lax.fori_loop(0, N_CHUNKS, chunk_body, 0)


def kernel(inp_tokens, emb_table):
    tokens = inp_tokens.reshape(-1).astype(jnp.int32)
    mesh = plsc.VectorSubcoreMesh(core_axis_name="c", subcore_axis_name="s")
    run = pl.kernel(
        _emb_kernel,
        mesh=mesh,
        out_type=jax.ShapeDtypeStruct((B_TOTAL, D), jnp.float32),
        scratch_types=[
            pltpu.VMEM((CH,), jnp.int32),
            pltpu.VMEM((CH, D), jnp.float32),
            pltpu.VMEM((CH + LANES,), jnp.float32),
            pltpu.SemaphoreType.DMA,
        ],
        compiler_params=pltpu.CompilerParams(use_tc_tiling_on_sc=False),
    )
    out = run(tokens, emb_table)
    return out.reshape(inp_tokens.shape + (D,))


# speedup vs baseline: 1.1030x; 1.1030x over previous
import jax
import jax.numpy as jnp
from jax import lax
from jax.experimental import pallas as pl
from jax.experimental.pallas import tpu as pltpu
from jax.experimental.pallas import tpu_sc as plsc

D = 64
SCALE = 8.0
LANES = 16

B_TOTAL = 4096 * 200
NW = 32
PER_W = B_TOTAL // NW
CH = 512
GATHER_W = 128
K = CH // GATHER_W
N_CHUNKS = PER_W // CH
NBUF = 2


def _emb_kernel(tok_hbm, table_hbm, out_hbm, idx_v, rows_v,
                sem_g0, sem_g1, sem_o0, sem_o1):
    nc = 2
    wid = lax.axis_index("s") * nc + lax.axis_index("c")
    base = wid * PER_W
    sem_g = (sem_g0, sem_g1)
    sem_o = (sem_o0, sem_o1)

    def fire(g, b):
        row0 = base + g * CH
        pltpu.sync_copy(tok_hbm.at[pl.ds(row0, CH)], idx_v.at[b])
        for j in range(K):
            pltpu.async_copy(
                table_hbm.at[idx_v.at[b, pl.ds(j * GATHER_W, GATHER_W)]],
                rows_v.at[b, pl.ds(j * GATHER_W, GATHER_W)],
                sem_g[b])

    def wait_gathers(b):
        pltpu.make_async_copy(
            out_hbm.at[pl.ds(0, CH)], rows_v.at[b], sem_g[b]).wait()

    def wait_out(b):
        pltpu.make_async_copy(
            rows_v.at[b], out_hbm.at[pl.ds(0, CH)], sem_o[b]).wait()

    def scale_chunk(b):
        def grp_body(q, c2):
            t = idx_v[b, pl.ds(q * LANES, LANES)]
            for r in range(LANES):
                row = q * LANES + r
                s = jnp.where(t[r] == 0, jnp.float32(0.0), jnp.float32(SCALE))
                sp = jnp.full((LANES,), s, jnp.float32)
                for c in range(D // LANES):
                    x = rows_v[b, row, pl.ds(c * LANES, LANES)]
                    rows_v[b, row, pl.ds(c * LANES, LANES)] = x * sp
            return c2

        lax.fori_loop(0, CH // LANES, grp_body, 0)

    def put(g, b):
        pltpu.async_copy(rows_v.at[b], out_hbm.at[pl.ds(base + g * CH, CH)],
                         sem_o[b])

    fire(0, 0)

    def chunk_body(g, carry):
        def stage(bb):
            @pl.when(g + 1 < N_CHUNKS)
            def _pref():
                @pl.when(g >= 1)
                def _w():
                    wait_out(1 - bb)
                fire(g + 1, 1 - bb)

            wait_gathers(bb)
            scale_chunk(bb)
            put(g, bb)

        @pl.when(lax.rem(g, 2) == 0)
        def _b0():
            stage(0)

        @pl.when(lax.rem(g, 2) == 1)
        def _b1():
            stage(1)

        return carry

    lax.fori_loop(0, N_CHUNKS, chunk_body, 0)
    wait_out((N_CHUNKS - 1) % 2)
    wait_out(N_CHUNKS % 2)


def kernel(inp_tokens, emb_table):
    tokens = inp_tokens.reshape(-1).astype(jnp.int32)
    mesh = plsc.VectorSubcoreMesh(core_axis_name="c", subcore_axis_name="s")
    run = pl.kernel(
        _emb_kernel,
        mesh=mesh,
        out_type=jax.ShapeDtypeStruct((B_TOTAL, D), jnp.float32),
        scratch_types=[
            pltpu.VMEM((NBUF, CH), jnp.int32),
            pltpu.VMEM((NBUF, CH, D), jnp.float32),
            pltpu.SemaphoreType.DMA,
            pltpu.SemaphoreType.DMA,
            pltpu.SemaphoreType.DMA,
            pltpu.SemaphoreType.DMA,
        ],
        compiler_params=pltpu.CompilerParams(use_tc_tiling_on_sc=False),
    )
    out = run(tokens, emb_table)
    return out.reshape(inp_tokens.shape + (D,))
